# Initial kernel scaffold; baseline (speedup 1.0000x reference)
import jax
import jax.numpy as jnp
from jax.experimental import pallas as pl


def kernel(states, We1, be1, We2, be2, ge, bbe, We3, be3, Wn1, bn1, Wn2, bn2, gn, bbn, Wn3, bn3, action):
    raise NotImplementedError("write your pallas kernel here")


# speedup vs baseline: 12.9179x; 12.9179x over previous
import jax
import jax.numpy as jnp
from jax import lax
from jax.experimental import pallas as pl
from jax.experimental.pallas import tpu as pltpu

NUM_OBJ = 10
D = 128
TB = 64
N = TB * NUM_OBJ


def _ln(x, g, b):
    m = jnp.mean(x, axis=-1, keepdims=True)
    xc = x - m
    v = jnp.mean(xc * xc, axis=-1, keepdims=True)
    return xc * lax.rsqrt(v + 1e-5) * g + b


def _fused(x_ref, a_ref, we1a_ref, we1b_ref, be1_ref, we2_ref, be2_ref,
           ge_ref, bbe_ref, we3_ref, be3_ref, wn1x_ref, wn1a_ref, wn1g_ref,
           bn1_ref, wn2_ref, bn2_ref, gn_ref, bbn_ref, wn3_ref, bn3_ref,
           o_ref):
    X = x_ref[...]
    P = jnp.dot(X, we1a_ref[...]) + be1_ref[...]
    Q = jnp.dot(X, we1b_ref[...])
    i_col = lax.broadcasted_iota(jnp.int32, (N, 1), 0) % NUM_OBJ
    S = jnp.zeros_like(X)
    for s in range(1, NUM_OBJ):
        up = pltpu.roll(Q, N - s, 0)
        down = pltpu.roll(Q, NUM_OBJ - s, 0)
        Qr = jnp.where(i_col < (NUM_OBJ - s), up, down)
        e = jax.nn.relu(P + Qr)
        e = jnp.dot(e, we2_ref[...]) + be2_ref[...]
        S = S + jax.nn.relu(_ln(e, ge_ref[...], bbe_ref[...]))
    agg = jnp.dot(S, we3_ref[...]) + (NUM_OBJ - 1) * be3_ref[...]
    h = (jnp.dot(X, wn1x_ref[...]) + jnp.dot(a_ref[...], wn1a_ref[...])
         + jnp.dot(agg, wn1g_ref[...]) + bn1_ref[...])
    h = jax.nn.relu(h)
    h = jnp.dot(h, wn2_ref[...]) + bn2_ref[...]
    h = jax.nn.relu(_ln(h, gn_ref[...], bbn_ref[...]))
    o_ref[...] = jnp.dot(h, wn3_ref[...]) + bn3_ref[...]


def kernel(states, We1, be1, We2, be2, ge, bbe, We3, be3,
           Wn1, bn1, Wn2, bn2, gn, bbn, Wn3, bn3, action):
    B, n, d = states.shape
    X = states.reshape(B * n, d)
    act_vec = jax.nn.one_hot(action, 4 * n, dtype=jnp.float32).reshape(B * n, 4)
    row2 = lambda v: v.reshape(1, -1)
    full = lambda shp: pl.BlockSpec(shp, lambda b: (0, 0))
    out = pl.pallas_call(
        _fused,
        grid=(B // TB,),
        in_specs=[
            pl.BlockSpec((N, d), lambda b: (b, 0)),
            pl.BlockSpec((N, 4), lambda b: (b, 0)),
            full((d, d)), full((d, d)), full((1, d)),
            full((d, d)), full((1, d)),
            full((1, d)), full((1, d)),
            full((d, d)), full((1, d)),
            full((d, d)), full((4, d)), full((d, d)),
            full((1, d)),
            full((d, d)), full((1, d)),
            full((1, d)), full((1, d)),
            full((d, d)), full((1, d)),
        ],
        out_specs=pl.BlockSpec((N, d), lambda b: (b, 0)),
        out_shape=jax.ShapeDtypeStruct((B * n, d), jnp.float32),
        compiler_params=pltpu.CompilerParams(
            dimension_semantics=("parallel",)),
    )(X, act_vec,
      We1[:d], We1[d:], row2(be1),
      We2, row2(be2), row2(ge), row2(bbe),
      We3, row2(be3),
      Wn1[:d], Wn1[d:d + 4], Wn1[d + 4:], row2(bn1),
      Wn2, row2(bn2), row2(gn), row2(bbn),
      Wn3, row2(bn3))
    return out.reshape(B, n, d)


# speedup vs baseline: 13.6615x; 1.0576x over previous
import jax
import jax.numpy as jnp
from jax import lax
from jax.experimental import pallas as pl
from jax.experimental.pallas import tpu as pltpu

NUM_OBJ = 10
D = 128
TB = 64
N = TB * NUM_OBJ


def _ln(x, g, b):
    m = jnp.mean(x, axis=-1, keepdims=True)
    xc = x - m
    v = jnp.mean(xc * xc, axis=-1, keepdims=True)
    return xc * lax.rsqrt(v + 1e-5) * g + b


def _fused(x_ref, a_ref, we1a_ref, we1b_ref, be1_ref, we2_ref, be2_ref,
           ge_ref, bbe_ref, we3_ref, be3_ref, wn1x_ref, wn1a_ref, wn1g_ref,
           bn1_ref, wn2_ref, bn2_ref, gn_ref, bbn_ref, wn3_ref, bn3_ref,
           o_ref):
    X = x_ref[...]
    P = jnp.dot(X, we1a_ref[...]) + be1_ref[...]
    Q = jnp.dot(X, we1b_ref[...])
    i_col = lax.broadcasted_iota(jnp.int32, (N, 1), 0) % NUM_OBJ
    Qd = pltpu.roll(Q, NUM_OBJ, 0)
    S = jnp.zeros_like(X)
    for s in range(1, NUM_OBJ):
        pre = jnp.where(i_col >= s, Q, Qd)
        Qr = pltpu.roll(pre, N - s, 0)
        e = jax.nn.relu(P + Qr)
        e = jnp.dot(e, we2_ref[...]) + be2_ref[...]
        S = S + jax.nn.relu(_ln(e, ge_ref[...], bbe_ref[...]))
    agg = jnp.dot(S, we3_ref[...]) + (NUM_OBJ - 1) * be3_ref[...]
    a = a_ref[...]
    rep = (lax.broadcasted_iota(jnp.int32, (N, TB), 0) // NUM_OBJ
           == lax.broadcasted_iota(jnp.int32, (N, TB), 1))
    arow = jnp.dot(rep.astype(jnp.float32), a)
    c = arow - 4.0 * i_col.astype(jnp.float32)
    actc = jnp.zeros_like(X)
    for k in range(4):
        actc = actc + jnp.where(c == float(k), wn1a_ref[k:k + 1, :], 0.0)
    h = (jnp.dot(X, wn1x_ref[...]) + actc
         + jnp.dot(agg, wn1g_ref[...]) + bn1_ref[...])
    h = jax.nn.relu(h)
    h = jnp.dot(h, wn2_ref[...]) + bn2_ref[...]
    h = jax.nn.relu(_ln(h, gn_ref[...], bbn_ref[...]))
    o_ref[...] = jnp.dot(h, wn3_ref[...]) + bn3_ref[...]


def kernel(states, We1, be1, We2, be2, ge, bbe, We3, be3,
           Wn1, bn1, Wn2, bn2, gn, bbn, Wn3, bn3, action):
    B, n, d = states.shape
    X = states.reshape(B * n, d)
    act_f = action.astype(jnp.float32).reshape(B, 1)
    row2 = lambda v: v.reshape(1, -1)
    full = lambda shp: pl.BlockSpec(shp, lambda b: (0, 0))
    out = pl.pallas_call(
        _fused,
        grid=(B // TB,),
        in_specs=[
            pl.BlockSpec((N, d), lambda b: (b, 0)),
            pl.BlockSpec((TB, 1), lambda b: (b, 0)),
            full((d, d)), full((d, d)), full((1, d)),
            full((d, d)), full((1, d)),
            full((1, d)), full((1, d)),
            full((d, d)), full((1, d)),
            full((d, d)), full((4, d)), full((d, d)),
            full((1, d)),
            full((d, d)), full((1, d)),
            full((1, d)), full((1, d)),
            full((d, d)), full((1, d)),
        ],
        out_specs=pl.BlockSpec((N, d), lambda b: (b, 0)),
        out_shape=jax.ShapeDtypeStruct((B * n, d), jnp.float32),
        compiler_params=pltpu.CompilerParams(
            dimension_semantics=("parallel",)),
    )(X, act_f,
      We1[:d], We1[d:], row2(be1),
      We2, row2(be2), row2(ge), row2(bbe),
      We3, row2(be3),
      Wn1[:d], Wn1[d:d + 4], Wn1[d + 4:], row2(bn1),
      Wn2, row2(bn2), row2(gn), row2(bbn),
      Wn3, row2(bn3))
    return out.reshape(B, n, d)


# speedup vs baseline: 13.6954x; 1.0025x over previous
import jax
import jax.numpy as jnp
from jax import lax
from jax.experimental import pallas as pl
from jax.experimental.pallas import tpu as pltpu

NUM_OBJ = 10
D = 128
TB = 64
N = TB * NUM_OBJ


def _ln(x, g, b):
    m = jnp.mean(x, axis=-1, keepdims=True)
    xc = x - m
    v = jnp.mean(xc * xc, axis=-1, keepdims=True)
    return xc * lax.rsqrt(v + 1e-5) * g + b


def _fused(x_ref, a_ref, we1_ref, be1_ref, we2_ref, be2_ref,
           ge_ref, bbe_ref, we3_ref, be3_ref, wn1_ref,
           bn1_ref, wn2_ref, bn2_ref, gn_ref, bbn_ref, wn3_ref, bn3_ref,
           o_ref):
    X = x_ref[...]
    P = jnp.dot(X, we1_ref[0:D, :]) + be1_ref[...]
    Q = jnp.dot(X, we1_ref[D:2 * D, :])
    i_col = lax.broadcasted_iota(jnp.int32, (N, 1), 0) % NUM_OBJ
    Qd = pltpu.roll(Q, NUM_OBJ, 0)
    S = jnp.zeros_like(X)
    for s in range(1, NUM_OBJ):
        pre = jnp.where(i_col >= s, Q, Qd)
        Qr = pltpu.roll(pre, N - s, 0)
        e = jax.nn.relu(P + Qr)
        e = jnp.dot(e, we2_ref[...]) + be2_ref[...]
        S = S + jax.nn.relu(_ln(e, ge_ref[...], bbe_ref[...]))
    agg = jnp.dot(S, we3_ref[...]) + (NUM_OBJ - 1) * be3_ref[...]
    a = a_ref[...].astype(jnp.float32)
    rep = (lax.broadcasted_iota(jnp.int32, (N, TB), 0) // NUM_OBJ
           == lax.broadcasted_iota(jnp.int32, (N, TB), 1))
    arow = jnp.dot(rep.astype(jnp.float32), a)
    c = arow - 4.0 * i_col.astype(jnp.float32)
    actc = jnp.zeros_like(X)
    for k in range(4):
        actc = actc + jnp.where(c == float(k), wn1_ref[D + k:D + k + 1, :], 0.0)
    h = (jnp.dot(X, wn1_ref[0:D, :]) + actc
         + jnp.dot(agg, wn1_ref[D + 4:D + 4 + D, :]) + bn1_ref[...])
    h = jax.nn.relu(h)
    h = jnp.dot(h, wn2_ref[...]) + bn2_ref[...]
    h = jax.nn.relu(_ln(h, gn_ref[...], bbn_ref[...]))
    o_ref[...] = jnp.dot(h, wn3_ref[...]) + bn3_ref[...]


def kernel(states, We1, be1, We2, be2, ge, bbe, We3, be3,
           Wn1, bn1, Wn2, bn2, gn, bbn, Wn3, bn3, action):
    B, n, d = states.shape
    X = states.reshape(B * n, d)
    act2 = action.reshape(B, 1)
    row2 = lambda v: v.reshape(1, -1)
    full = lambda shp: pl.BlockSpec(shp, lambda b: (0, 0))
    out = pl.pallas_call(
        _fused,
        grid=(B // TB,),
        in_specs=[
            pl.BlockSpec((N, d), lambda b: (b, 0)),
            pl.BlockSpec((TB, 1), lambda b: (b, 0)),
            full((2 * d, d)), full((1, d)),
            full((d, d)), full((1, d)),
            full((1, d)), full((1, d)),
            full((d, d)), full((1, d)),
            full((d + 4 + d, d)),
            full((1, d)),
            full((d, d)), full((1, d)),
            full((1, d)), full((1, d)),
            full((d, d)), full((1, d)),
        ],
        out_specs=pl.BlockSpec((N, d), lambda b: (b, 0)),
        out_shape=jax.ShapeDtypeStruct((B * n, d), jnp.float32),
        compiler_params=pltpu.CompilerParams(
            dimension_semantics=("parallel",)),
    )(X, act2,
      We1, row2(be1),
      We2, row2(be2), row2(ge), row2(bbe),
      We3, row2(be3),
      Wn1, row2(bn1),
      Wn2, row2(bn2), row2(gn), row2(bbn),
      Wn3, row2(bn3))
    return out.reshape(B, n, d)
